# baseline (speedup 1.0000x reference)
import functools

import jax
import jax.numpy as jnp
import numpy as np
from jax import lax
from jax.experimental import pallas as pl
from jax.experimental.pallas import tpu as pltpu
from jax.experimental.pallas import tpu_sc as plsc

B, S, H = 4, 2048, 1024
HH = H // 2
TOK = B * S
EPS = 1e-12

NC, NS = 2, 16
NW = NC * NS
CH = 32
NBUF = 2

ROWS = 2048
NBLK = TOK // ROWS
POS_BLKS = S // ROWS

_RND = np.uint32(0x7FFF)
_ONE = np.uint32(1)
_HI = np.uint32(0xFFFF0000)


def _bf16_word(a, b):
    au = lax.bitcast_convert_type(a, jnp.uint32)
    bu = lax.bitcast_convert_type(b, jnp.uint32)
    au = au + (_RND + ((au >> 16) & _ONE))
    bu = bu + (_RND + ((bu >> 16) & _ONE))
    return (au >> 16) | (bu & _HI)


def _make_sc_gather(tokc):
    tpw = tokc // NW
    nchunk = tpw // CH

    def body(ids_hbm, table_hbm, out_hbm, idx_v, fbuf, pbuf, gsem, ssem):
        wid = lax.axis_index("s") * NC + lax.axis_index("c")
        base = wid * tpw
        pltpu.sync_copy(ids_hbm.at[pl.ds(base, tpw)], idx_v)

        store_done = [None] * NBUF

        def start_gather(c):
            bi = c % NBUF
            if store_done[bi] is not None:
                store_done[bi].wait()
            return pltpu.async_copy(
                table_hbm.at[idx_v.at[pl.ds(c * CH, CH)]], fbuf.at[bi], gsem)

        def convert(bi):
            def row(r, carry):
                for j in range(HH // 16):
                    a = fbuf[bi, r, pl.ds(j * 16, 16)]
                    b = fbuf[bi, r, pl.ds(HH + j * 16, 16)]
                    pbuf[bi, r, pl.ds(j * 16, 16)] = _bf16_word(a, b)
                return carry
            lax.fori_loop(0, CH, row, 0)

        gcur = start_gather(0)
        for c in range(nchunk):
            bi = c % NBUF
            gnext = start_gather(c + 1) if c + 1 < nchunk else None
            gcur.wait()
            convert(bi)
            store_done[bi] = pltpu.async_copy(
                pbuf.at[bi], out_hbm.at[pl.ds(base + c * CH, CH)], ssem)
            gcur = gnext
        for d in store_done:
            if d is not None:
                d.wait()

    return pl.kernel(
        body,
        out_type=jax.ShapeDtypeStruct((tokc, HH), jnp.uint32),
        mesh=plsc.VectorSubcoreMesh(core_axis_name="c", subcore_axis_name="s"),
        scratch_types=[
            pltpu.VMEM((tpw,), jnp.int32),
            pltpu.VMEM((NBUF, CH, H), jnp.float32),
            pltpu.VMEM((NBUF, CH, HH), jnp.uint32),
            pltpu.SemaphoreType.DMA,
            pltpu.SemaphoreType.DMA,
        ],
    )


_sc_gather = _make_sc_gather(TOK)


def _ln_body(tt_ref, g_ref, pos_ref, type_ref, gamma_ref, beta_ref, out_ref):
    u = g_ref[...]
    wlo = lax.bitcast_convert_type(u << 16, jnp.float32)
    whi = lax.bitcast_convert_type(u & _HI, jnp.float32)
    f = tt_ref[0, 0, :].astype(jnp.float32).reshape(ROWS, 1)
    t0 = type_ref[0, :].reshape(1, H)
    t1 = type_ref[1, :].reshape(1, H)
    dt = t1 - t0
    xlo = wlo + pos_ref[:, :HH] + t0[:, :HH] + f * dt[:, :HH]
    xhi = whi + pos_ref[:, HH:] + t0[:, HH:] + f * dt[:, HH:]
    s = jnp.sum(xlo, axis=-1, keepdims=True) \
        + jnp.sum(xhi, axis=-1, keepdims=True)
    mean = s * (1.0 / H)
    clo = xlo - mean
    chi = xhi - mean
    v = jnp.sum(clo * clo, axis=-1, keepdims=True) \
        + jnp.sum(chi * chi, axis=-1, keepdims=True)
    rstd = lax.rsqrt(v * (1.0 / H) + EPS)
    gm = gamma_ref[0, :].reshape(1, H)
    bt = beta_ref[0, :].reshape(1, H)
    out_ref[:, :HH] = clo * rstd * gm[:, :HH] + bt[:, :HH]
    out_ref[:, HH:] = chi * rstd * gm[:, HH:] + bt[:, HH:]


_ln_call = pl.pallas_call(
    _ln_body,
    grid=(POS_BLKS, B),
    in_specs=[
        pl.BlockSpec((1, 1, ROWS), lambda p, b: (b * POS_BLKS + p, 0, 0)),
        pl.BlockSpec((ROWS, HH), lambda p, b: (b * POS_BLKS + p, 0)),
        pl.BlockSpec((ROWS, H), lambda p, b: (p, 0)),
        pl.BlockSpec((2, H), lambda p, b: (0, 0)),
        pl.BlockSpec((1, H), lambda p, b: (0, 0)),
        pl.BlockSpec((1, H), lambda p, b: (0, 0)),
    ],
    out_specs=pl.BlockSpec((ROWS, H),
                           lambda p, b: (b * POS_BLKS + p, 0)),
    out_shape=jax.ShapeDtypeStruct((TOK, H), jnp.float32),
)


def kernel(input_ids, token_type_ids, word_table, pos_table, type_table,
           gamma, beta):
    ids = input_ids.reshape(TOK).astype(jnp.int32)
    tt3 = token_type_ids.reshape(NBLK, 1, ROWS).astype(jnp.int32)
    packed = _sc_gather(ids, word_table)
    out = _ln_call(tt3, packed, pos_table, type_table,
                   gamma.reshape(1, H), beta.reshape(1, H))
    return out.reshape(B, S, H)


# speedup vs baseline: 1.6454x; 1.6454x over previous
import functools

import jax
import jax.numpy as jnp
from jax import lax
from jax.experimental import pallas as pl
from jax.experimental.pallas import tpu as pltpu
from jax.experimental.pallas import tpu_sc as plsc

B, S, H = 4, 2048, 1024
TOK = B * S
EPS = 1e-12

NC, NS = 2, 16
NW = NC * NS
CH = 32
NBUF = 3

CHUNK_SEQS = [4]

ROWS = 2048
NBLK = TOK // ROWS
POS_BLKS = S // ROWS


def _make_sc_gather(tokc):
    tpw = tokc // NW
    nchunk = tpw // CH

    def body(ids_hbm, table_hbm, out_hbm, idx_v, bufs, gsem, ssem):
        wid = lax.axis_index("s") * NC + lax.axis_index("c")
        base = wid * tpw
        pltpu.sync_copy(ids_hbm.at[pl.ds(base, tpw)], idx_v)

        store_done = [None] * NBUF

        def start_gather(c):
            bi = c % NBUF
            if store_done[bi] is not None:
                store_done[bi].wait()
            return pltpu.async_copy(
                table_hbm.at[idx_v.at[pl.ds(c * CH, CH)]], bufs.at[bi], gsem)

        gcur = start_gather(0)
        for c in range(nchunk):
            bi = c % NBUF
            gnext = start_gather(c + 1) if c + 1 < nchunk else None
            gcur.wait()
            store_done[bi] = pltpu.async_copy(
                bufs.at[bi], out_hbm.at[pl.ds(base + c * CH, CH)], ssem)
            gcur = gnext
        for d in store_done:
            if d is not None:
                d.wait()

    return pl.kernel(
        body,
        out_type=jax.ShapeDtypeStruct((tokc, H), jnp.float32),
        mesh=plsc.VectorSubcoreMesh(core_axis_name="c", subcore_axis_name="s"),
        scratch_types=[
            pltpu.VMEM((tpw,), jnp.int32),
            pltpu.VMEM((NBUF, CH, H), jnp.float32),
            pltpu.SemaphoreType.DMA,
            pltpu.SemaphoreType.DMA,
        ],
    )


def _ln_math(tt_ref, g_ref, pos_ref, type_ref, gamma_ref, beta_ref, out_ref):
    x = g_ref[...] + pos_ref[...]
    f = tt_ref[0, 0, :].astype(jnp.float32).reshape(ROWS, 1)
    t0 = type_ref[0, :].reshape(1, H)
    t1 = type_ref[1, :].reshape(1, H)
    x = x + t0 + f * (t1 - t0)
    mean = jnp.mean(x, axis=-1, keepdims=True)
    xc = x - mean
    var = jnp.mean(xc * xc, axis=-1, keepdims=True)
    rstd = lax.rsqrt(var + EPS)
    out_ref[...] = xc * rstd * gamma_ref[0, :].reshape(1, H) \
        + beta_ref[0, :].reshape(1, H)


def _ln_first_body(tt_ref, g_ref, pos_ref, type_ref, gamma_ref, beta_ref,
                   out_ref):
    _ln_math(tt_ref, g_ref, pos_ref, type_ref, gamma_ref, beta_ref, out_ref)


def _ln_next_body(tt_ref, g_ref, pos_ref, type_ref, gamma_ref, beta_ref,
                  prev_ref, out_ref):
    del prev_ref
    _ln_math(tt_ref, g_ref, pos_ref, type_ref, gamma_ref, beta_ref, out_ref)


def _make_ln(first, base_blk, bc):
    in_specs = [
        pl.BlockSpec((1, 1, ROWS),
                     lambda p, b: (base_blk + b * POS_BLKS + p, 0, 0)),
        pl.BlockSpec((ROWS, H), lambda p, b: (b * POS_BLKS + p, 0)),
        pl.BlockSpec((ROWS, H), lambda p, b: (p, 0)),
        pl.BlockSpec((2, H), lambda p, b: (0, 0)),
        pl.BlockSpec((1, H), lambda p, b: (0, 0)),
        pl.BlockSpec((1, H), lambda p, b: (0, 0)),
    ]
    kwargs = {}
    body = _ln_first_body
    if not first:
        in_specs.append(pl.BlockSpec(memory_space=pltpu.MemorySpace.HBM))
        kwargs["input_output_aliases"] = {6: 0}
        body = _ln_next_body
    return pl.pallas_call(
        body,
        grid=(POS_BLKS, bc),
        in_specs=in_specs,
        out_specs=pl.BlockSpec((ROWS, H),
                               lambda p, b: (base_blk + b * POS_BLKS + p, 0)),
        out_shape=jax.ShapeDtypeStruct((TOK, H), jnp.float32),
        **kwargs,
    )


_sc_calls = [_make_sc_gather(bc * S) for bc in CHUNK_SEQS]
_ln_calls = []
_blk = 0
for _i, _bc in enumerate(CHUNK_SEQS):
    _ln_calls.append(_make_ln(_i == 0, _blk, _bc))
    _blk += _bc * S // ROWS


def kernel(input_ids, token_type_ids, word_table, pos_table, type_table,
           gamma, beta):
    ids = input_ids.reshape(TOK).astype(jnp.int32)
    tt3 = token_type_ids.reshape(NBLK, 1, ROWS).astype(jnp.int32)
    g2 = gamma.reshape(1, H)
    b2 = beta.reshape(1, H)
    gathered = []
    base = 0
    for i, bc in enumerate(CHUNK_SEQS):
        gathered.append(_sc_calls[i](ids[base:base + bc * S], word_table))
        base += bc * S
    out = _ln_calls[0](tt3, gathered[0], pos_table, type_table, g2, b2)
    for i in range(1, len(CHUNK_SEQS)):
        out = _ln_calls[i](tt3, gathered[i], pos_table, type_table, g2, b2,
                           out)
    return out.reshape(B, S, H)
